# baseline (speedup 1.0000x reference)
import functools

import jax
import jax.numpy as jnp
import numpy as np
from jax import lax
from jax.experimental import pallas as pl
from jax.experimental.pallas import tpu as pltpu
from jax.experimental.pallas import tpu_sc as plsc

_N_POINTS = 262144
_N_LEVELS = 16
_LOG2_HASH = 19
_TABLE = 2 ** _LOG2_HASH
_MASK = _TABLE - 1
_P1 = np.int32(np.uint32(2654435761).view(np.int32))
_P2 = np.int32(805459861)
_BASE_RES = 16.0
_FINEST_RES = 512.0
_B = float(np.exp((np.log(_FINEST_RES) - np.log(_BASE_RES)) / (_N_LEVELS - 1)))
_GS = [float(np.float32(1.0) / np.float32(np.floor(_BASE_RES * _B ** i)))
       for i in range(_N_LEVELS)]

_NC, _NS = 2, 16
_NW = _NC * _NS
_PTS_PER_W = _N_POINTS // _NW
_CHUNK = 512
_NCHUNK = _PTS_PER_W // _CHUNK
_NG = _CHUNK // 16
_STREAM = 512
_NSTREAM = 8 * _CHUNK // _STREAM
_RPC = _CHUNK // _STREAM
_SPG = _STREAM // 16

_SCRATCH = [
    pltpu.VMEM((3 * _CHUNK,), jnp.float32),
    pltpu.VMEM((3, _CHUNK), jnp.float32),
    pltpu.VMEM((_NSTREAM, _STREAM), jnp.int32),
    pltpu.VMEM((_NSTREAM, _STREAM), jnp.int32),
    pltpu.VMEM((_NSTREAM, _STREAM, 8), jnp.float32),
    pltpu.VMEM((2 * _N_LEVELS, _CHUNK), jnp.float32),
    pltpu.VMEM((16,), jnp.float32),
    pltpu.SemaphoreType.DMA,
]


def _sc_body(xyz_hbm, emb_hbm, lw_hbm, out_hbm,
             xyz_v, wbuf, idx_v, sub_v, rows_v, out_v, gate_v, sem):
    wid = lax.axis_index("c") * _NS + lax.axis_index("s")
    iota = lax.iota(jnp.int32, 16)

    pltpu.sync_copy(lw_hbm, gate_v)
    lw = gate_v[...]
    gate_v[...] = 1.0 / (1.0 + jnp.exp(-lw))

    def do_chunk(ch, carry):
        base = wid * _PTS_PER_W + ch * _CHUNK
        for d in range(3):
            pltpu.sync_copy(xyz_hbm.at[pl.ds(d * _N_POINTS + base, _CHUNK)],
                            xyz_v.at[pl.ds(d * _CHUNK, _CHUNK)])

        for l in range(_N_LEVELS):
            gs = np.float32(_GS[l])
            lvl_off = np.int32(l * _TABLE)

            def pass_a(jj, c, gs=gs, lvl_off=lvl_off):
                p0 = pl.multiple_of(jj * 16, 16)
                x = xyz_v[pl.ds(p0, 16)]
                y = xyz_v[pl.ds(_CHUNK + p0, 16)]
                z = xyz_v[pl.ds(2 * _CHUNK + p0, 16)]
                xc = jnp.minimum(jnp.maximum(x, 0.0), 1.0)
                yc = jnp.minimum(jnp.maximum(y, 0.0), 1.0)
                zc = jnp.minimum(jnp.maximum(z, 0.0), 1.0)
                bx = (xc / gs).astype(jnp.int32)
                by = (yc / gs).astype(jnp.int32)
                bz = (zc / gs).astype(jnp.int32)
                vminx = bx.astype(jnp.float32) * gs
                vminy = by.astype(jnp.float32) * gs
                vminz = bz.astype(jnp.float32) * gs
                wbuf[0, pl.ds(p0, 16)] = (x - vminx) / ((vminx + gs) - vminx)
                wbuf[1, pl.ds(p0, 16)] = (y - vminy) / ((vminy + gs) - vminy)
                wbuf[2, pl.ds(p0, 16)] = (z - vminz) / ((vminz + gs) - vminz)
                row = jj // _SPG
                col = pl.multiple_of((jj % _SPG) * 16, 16)
                t0, t0p = bx, bx + 1
                t1, t1p = by * _P1, by * _P1 + _P1
                t2, t2p = bz * _P2, bz * _P2 + _P2
                for c8, (a, b, d) in enumerate(
                        ((t0, t1, t2), (t0, t1, t2p), (t0, t1p, t2),
                         (t0, t1p, t2p), (t0p, t1, t2), (t0p, t1, t2p),
                         (t0p, t1p, t2), (t0p, t1p, t2p))):
                    h = ((a ^ b ^ d) & _MASK) + lvl_off
                    idx_v[c8 * _RPC + row, pl.ds(col, 16)] = (
                        lax.shift_right_logical(h, 2))
                    sub_v[c8 * _RPC + row, pl.ds(col, 16)] = (h & 3) * 2
                return c

            lax.fori_loop(0, _NG, pass_a, 0, unroll=False)

            def fire(i, c):
                pltpu.make_async_copy(
                    emb_hbm.at[idx_v.at[i]], rows_v.at[i], sem).start()
                return c

            lax.fori_loop(0, _NSTREAM, fire, 0, unroll=False)

            def drain(i, c):
                pltpu.make_async_copy(
                    emb_hbm.at[idx_v.at[i]], rows_v.at[i], sem).wait()
                return c

            lax.fori_loop(0, _NSTREAM, drain, 0, unroll=False)

            gate = plsc.load_gather(gate_v, [jnp.full((16,), l, jnp.int32)])

            def pass_b(jj, c, gate=gate, l=l):
                p0 = pl.multiple_of(jj * 16, 16)
                row0 = jj // _SPG
                col = (jj % _SPG) * 16 + iota
                w0 = wbuf[0, pl.ds(p0, 16)]
                w1 = wbuf[1, pl.ds(p0, 16)]
                w2 = wbuf[2, pl.ds(p0, 16)]
                for feat in range(2):
                    e = []
                    for c8 in range(8):
                        sub = sub_v[c8 * _RPC + row0, pl.ds(
                            pl.multiple_of((jj % _SPG) * 16, 16), 16)]
                        e.append(plsc.load_gather(
                            rows_v,
                            [jnp.full((16,), c8 * _RPC, jnp.int32) + row0,
                             col, sub + feat]))
                    c00 = e[0] * (1 - w0) + e[4] * w0
                    c01 = e[1] * (1 - w0) + e[5] * w0
                    c10 = e[2] * (1 - w0) + e[6] * w0
                    c11 = e[3] * (1 - w0) + e[7] * w0
                    c0 = c00 * (1 - w1) + c10 * w1
                    c1 = c01 * (1 - w1) + c11 * w1
                    val = (c0 * (1 - w2) + c1 * w2) * gate
                    out_v[2 * l + feat, pl.ds(p0, 16)] = val
                return c

            lax.fori_loop(0, _NG, pass_b, 0, unroll=False)

        pltpu.sync_copy(out_v, out_hbm.at[:, pl.ds(base, _CHUNK)])
        return carry

    lax.fori_loop(0, _NCHUNK, do_chunk, 0, unroll=False)


def _prep(inputs, embeddings, level_weights):
    return (inputs.T.reshape(-1),
            embeddings.reshape(_N_LEVELS * _TABLE // 4, 8),
            level_weights)


@jax.jit
def _encode(xyz_flat, emb2, level_weights):
    mesh = plsc.VectorSubcoreMesh(core_axis_name="c", subcore_axis_name="s")
    f = pl.kernel(
        _sc_body,
        out_type=jax.ShapeDtypeStruct((2 * _N_LEVELS, _N_POINTS), jnp.float32),
        mesh=mesh,
        compiler_params=pltpu.CompilerParams(
            needs_layout_passes=False, use_tc_tiling_on_sc=False),
        scratch_types=_SCRATCH,
    )
    return f(xyz_flat, emb2, level_weights)


def kernel(inputs, embeddings, level_weights):
    out_t = _encode(*_prep(inputs, embeddings, level_weights))
    eye = jnp.eye(2 * _N_LEVELS, dtype=jnp.float32)
    return jax.numpy.einsum("kc,kf->cf", out_t, eye,
                            precision=jax.lax.Precision.HIGHEST)


# speedup vs baseline: 3.9376x; 3.9376x over previous
import functools

import jax
import jax.numpy as jnp
import numpy as np
from jax import lax
from jax.experimental import pallas as pl
from jax.experimental.pallas import tpu as pltpu
from jax.experimental.pallas import tpu_sc as plsc

_N_POINTS = 262144
_N_LEVELS = 16
_LOG2_HASH = 19
_TABLE = 2 ** _LOG2_HASH
_MASK = _TABLE - 1
_P1 = np.int32(np.uint32(2654435761).view(np.int32))
_P2 = np.int32(805459861)
_BASE_RES = 16.0
_FINEST_RES = 512.0
_B = float(np.exp((np.log(_FINEST_RES) - np.log(_BASE_RES)) / (_N_LEVELS - 1)))
_GS = [float(np.float32(1.0) / np.float32(np.floor(_BASE_RES * _B ** i)))
       for i in range(_N_LEVELS)]

_NC, _NS = 2, 16
_NW = _NC * _NS
_PTS_PER_W = _N_POINTS // _NW
_CHUNK = 512
_NCHUNK = _PTS_PER_W // _CHUNK
_NG = _CHUNK // 16
_STREAM = 512
_NSTREAM = 8 * _CHUNK // _STREAM
_RPC = _CHUNK // _STREAM
_SPG = _STREAM // 16

_SCRATCH = [
    pltpu.VMEM((3 * _CHUNK,), jnp.float32),
    pltpu.VMEM((3, _CHUNK), jnp.float32),
    pltpu.VMEM((_NSTREAM, _STREAM), jnp.int32),
    pltpu.VMEM((_NSTREAM, _STREAM), jnp.int32),
    pltpu.VMEM((_NSTREAM, _STREAM, 8), jnp.float32),
    pltpu.VMEM((2 * _N_LEVELS, _CHUNK), jnp.float32),
    pltpu.VMEM((16,), jnp.float32),
    pltpu.SemaphoreType.DMA,
]


def _sc_body(xyz_hbm, emb_hbm, lw_hbm, out_hbm,
             xyz_v, wbuf, idx_v, sub_v, rows_v, out_v, gate_v, sem):
    wid = lax.axis_index("c") * _NS + lax.axis_index("s")
    iota = lax.iota(jnp.int32, 16)

    pltpu.sync_copy(lw_hbm, gate_v)
    lw = gate_v[...]
    gate_v[...] = 1.0 / (1.0 + jnp.exp(-lw))

    def do_chunk(ch, carry):
        base = wid * _PTS_PER_W + ch * _CHUNK
        for d in range(3):
            pltpu.sync_copy(xyz_hbm.at[pl.ds(d * _N_POINTS + base, _CHUNK)],
                            xyz_v.at[pl.ds(d * _CHUNK, _CHUNK)])

        for l in range(_N_LEVELS):
            gs = np.float32(_GS[l])
            lvl_off = np.int32(l * _TABLE)

            def pass_a(jj, c, gs=gs, lvl_off=lvl_off):
                p0 = pl.multiple_of(jj * 16, 16)
                x = xyz_v[pl.ds(p0, 16)]
                y = xyz_v[pl.ds(_CHUNK + p0, 16)]
                z = xyz_v[pl.ds(2 * _CHUNK + p0, 16)]
                xc = jnp.minimum(jnp.maximum(x, 0.0), 1.0)
                yc = jnp.minimum(jnp.maximum(y, 0.0), 1.0)
                zc = jnp.minimum(jnp.maximum(z, 0.0), 1.0)
                bx = (xc / gs).astype(jnp.int32)
                by = (yc / gs).astype(jnp.int32)
                bz = (zc / gs).astype(jnp.int32)
                vminx = bx.astype(jnp.float32) * gs
                vminy = by.astype(jnp.float32) * gs
                vminz = bz.astype(jnp.float32) * gs
                wbuf[0, pl.ds(p0, 16)] = (x - vminx) / ((vminx + gs) - vminx)
                wbuf[1, pl.ds(p0, 16)] = (y - vminy) / ((vminy + gs) - vminy)
                wbuf[2, pl.ds(p0, 16)] = (z - vminz) / ((vminz + gs) - vminz)
                row = jj // _SPG
                col = pl.multiple_of((jj % _SPG) * 16, 16)
                t0, t0p = bx, bx + 1
                t1, t1p = by * _P1, by * _P1 + _P1
                t2, t2p = bz * _P2, bz * _P2 + _P2
                for c8, (a, b, d) in enumerate(
                        ((t0, t1, t2), (t0, t1, t2p), (t0, t1p, t2),
                         (t0, t1p, t2p), (t0p, t1, t2), (t0p, t1, t2p),
                         (t0p, t1p, t2), (t0p, t1p, t2p))):
                    h = ((a ^ b ^ d) & _MASK) + lvl_off
                    idx_v[c8 * _RPC + row, pl.ds(col, 16)] = (
                        lax.shift_right_logical(h, 2))
                    sub_v[c8 * _RPC + row, pl.ds(col, 16)] = (h & 3) * 2
                return c

            lax.fori_loop(0, _NG, pass_a, 0, unroll=False)

            def fire(i, c):
                pltpu.make_async_copy(
                    emb_hbm.at[idx_v.at[i]], rows_v.at[i], sem).start()
                return c

            lax.fori_loop(0, _NSTREAM, fire, 0, unroll=False)

            def drain(i, c):
                pltpu.make_async_copy(
                    emb_hbm.at[idx_v.at[i]], rows_v.at[i], sem).wait()
                return c

            lax.fori_loop(0, _NSTREAM, drain, 0, unroll=False)

            gate = plsc.load_gather(gate_v, [jnp.full((16,), l, jnp.int32)])

            def pass_b(jj, c, gate=gate, l=l):
                p0 = pl.multiple_of(jj * 16, 16)
                row0 = jj // _SPG
                col = (jj % _SPG) * 16 + iota
                w0 = wbuf[0, pl.ds(p0, 16)]
                w1 = wbuf[1, pl.ds(p0, 16)]
                w2 = wbuf[2, pl.ds(p0, 16)]
                for feat in range(2):
                    e = []
                    for c8 in range(8):
                        sub = sub_v[c8 * _RPC + row0, pl.ds(
                            pl.multiple_of((jj % _SPG) * 16, 16), 16)]
                        e.append(plsc.load_gather(
                            rows_v,
                            [jnp.full((16,), c8 * _RPC, jnp.int32) + row0,
                             col, sub + feat]))
                    c00 = e[0] * (1 - w0) + e[4] * w0
                    c01 = e[1] * (1 - w0) + e[5] * w0
                    c10 = e[2] * (1 - w0) + e[6] * w0
                    c11 = e[3] * (1 - w0) + e[7] * w0
                    c0 = c00 * (1 - w1) + c10 * w1
                    c1 = c01 * (1 - w1) + c11 * w1
                    val = (c0 * (1 - w2) + c1 * w2) * gate
                    out_v[2 * l + feat, pl.ds(p0, 16)] = val
                return c

            lax.fori_loop(0, _NG, pass_b, 0, unroll=False)

        pltpu.sync_copy(out_v, out_hbm.at[:, pl.ds(base, _CHUNK)])
        return carry

    lax.fori_loop(0, _NCHUNK, do_chunk, 0, unroll=False)


_SEL = np.zeros((4, 2, 8), np.float32)
for _c in range(4):
    for _f in range(2):
        _SEL[_c, _f, 2 * _c + _f] = 1.0


def _prep(inputs, embeddings, level_weights):
    e4 = embeddings.reshape(_N_LEVELS, _TABLE // 4, 4, 2)
    emb8 = jnp.einsum("lrcf,cfj->lrj", e4, jnp.asarray(_SEL),
                      precision=jax.lax.Precision.HIGHEST)
    return (inputs.T.reshape(-1),
            emb8.reshape(_N_LEVELS * _TABLE // 4, 8),
            level_weights)


@jax.jit
def _encode(xyz_flat, emb2, level_weights):
    mesh = plsc.VectorSubcoreMesh(core_axis_name="c", subcore_axis_name="s")
    f = pl.kernel(
        _sc_body,
        out_type=jax.ShapeDtypeStruct((2 * _N_LEVELS, _N_POINTS), jnp.float32),
        mesh=mesh,
        compiler_params=pltpu.CompilerParams(
            needs_layout_passes=False, use_tc_tiling_on_sc=False),
        scratch_types=_SCRATCH,
    )
    return f(xyz_flat, emb2, level_weights)


def kernel(inputs, embeddings, level_weights):
    out_t = _encode(*_prep(inputs, embeddings, level_weights))
    eye = jnp.eye(2 * _N_LEVELS, dtype=jnp.float32)
    return jax.numpy.einsum("kc,kf->cf", out_t, eye,
                            precision=jax.lax.Precision.HIGHEST)


# speedup vs baseline: 5.0454x; 1.2813x over previous
import functools

import jax
import jax.numpy as jnp
import numpy as np
from jax import lax
from jax.experimental import pallas as pl
from jax.experimental.pallas import tpu as pltpu
from jax.experimental.pallas import tpu_sc as plsc

_N_POINTS = 262144
_N_LEVELS = 16
_LOG2_HASH = 19
_TABLE = 2 ** _LOG2_HASH
_MASK = _TABLE - 1
_P1 = np.int32(np.uint32(2654435761).view(np.int32))
_P2 = np.int32(805459861)
_BASE_RES = 16.0
_FINEST_RES = 512.0
_B = float(np.exp((np.log(_FINEST_RES) - np.log(_BASE_RES)) / (_N_LEVELS - 1)))
_GS = [float(np.float32(1.0) / np.float32(np.floor(_BASE_RES * _B ** i)))
       for i in range(_N_LEVELS)]

_NC, _NS = 2, 16
_NW = _NC * _NS
_PTS_PER_W = _N_POINTS // _NW
_CHUNK = 512
_NCHUNK = _PTS_PER_W // _CHUNK
_NG = _CHUNK // 16
_STREAM = 512
_NSTREAM = 8 * _CHUNK // _STREAM
_RPC = _CHUNK // _STREAM
_SPG = _STREAM // 16

_SCRATCH = [
    pltpu.VMEM((3 * _CHUNK,), jnp.float32),
    pltpu.VMEM((2, 3, _CHUNK), jnp.float32),
    pltpu.VMEM((2, _NSTREAM, _STREAM), jnp.int32),
    pltpu.VMEM((2, _NSTREAM, _STREAM), jnp.int32),
    pltpu.VMEM((2, _NSTREAM, _STREAM, 8), jnp.float32),
    pltpu.VMEM((2 * _N_LEVELS, _CHUNK), jnp.float32),
    pltpu.VMEM((16,), jnp.float32),
    pltpu.SemaphoreType.DMA,
    pltpu.SemaphoreType.DMA,
]


def _sc_body(xyz_hbm, emb_hbm, lw_hbm, out_hbm,
             xyz_v, wbuf, idx_v, sub_v, rows_v, out_v, gate_v, sem0, sem1):
    wid = lax.axis_index("c") * _NS + lax.axis_index("s")
    iota = lax.iota(jnp.int32, 16)
    sems = (sem0, sem1)

    pltpu.sync_copy(lw_hbm, gate_v)
    lw = gate_v[...]
    gate_v[...] = 1.0 / (1.0 + jnp.exp(-lw))

    def pass_a(l, bf):
        gs = np.float32(_GS[l])
        lvl_off = np.int32(l * _TABLE)

        def body(jj, c):
            p0 = pl.multiple_of(jj * 16, 16)
            pidx = (p0 + iota) * 3
            x = plsc.load_gather(xyz_v, [pidx])
            y = plsc.load_gather(xyz_v, [pidx + 1])
            z = plsc.load_gather(xyz_v, [pidx + 2])
            xc = jnp.minimum(jnp.maximum(x, 0.0), 1.0)
            yc = jnp.minimum(jnp.maximum(y, 0.0), 1.0)
            zc = jnp.minimum(jnp.maximum(z, 0.0), 1.0)
            bx = (xc / gs).astype(jnp.int32)
            by = (yc / gs).astype(jnp.int32)
            bz = (zc / gs).astype(jnp.int32)
            vminx = bx.astype(jnp.float32) * gs
            vminy = by.astype(jnp.float32) * gs
            vminz = bz.astype(jnp.float32) * gs
            wbuf[bf, 0, pl.ds(p0, 16)] = (x - vminx) / ((vminx + gs) - vminx)
            wbuf[bf, 1, pl.ds(p0, 16)] = (y - vminy) / ((vminy + gs) - vminy)
            wbuf[bf, 2, pl.ds(p0, 16)] = (z - vminz) / ((vminz + gs) - vminz)
            row = jj // _SPG
            col = pl.multiple_of((jj % _SPG) * 16, 16)
            t0, t0p = bx, bx + 1
            t1, t1p = by * _P1, by * _P1 + _P1
            t2, t2p = bz * _P2, bz * _P2 + _P2
            for c8, (a, b, d) in enumerate(
                    ((t0, t1, t2), (t0, t1, t2p), (t0, t1p, t2),
                     (t0, t1p, t2p), (t0p, t1, t2), (t0p, t1, t2p),
                     (t0p, t1p, t2), (t0p, t1p, t2p))):
                h = ((a ^ b ^ d) & _MASK) + lvl_off
                idx_v[bf, c8 * _RPC + row, pl.ds(col, 16)] = (
                    lax.shift_right_logical(h, 2))
                sub_v[bf, c8 * _RPC + row, pl.ds(col, 16)] = (h & 3) * 2
            return c

        lax.fori_loop(0, _NG, body, 0, unroll=False)

    def fire(bf):
        def body(i, c):
            pltpu.make_async_copy(
                emb_hbm.at[idx_v.at[bf, i]], rows_v.at[bf, i],
                sems[bf]).start()
            return c
        lax.fori_loop(0, _NSTREAM, body, 0, unroll=False)

    def drain(bf):
        def body(i, c):
            pltpu.make_async_copy(
                emb_hbm.at[idx_v.at[bf, i]], rows_v.at[bf, i],
                sems[bf]).wait()
            return c
        lax.fori_loop(0, _NSTREAM, body, 0, unroll=False)

    def pass_b(l, bf):
        gate = plsc.load_gather(gate_v, [jnp.full((16,), l, jnp.int32)])
        rows_b = rows_v.at[bf]
        sub_b = sub_v.at[bf]

        def body(jj, c):
            p0 = pl.multiple_of(jj * 16, 16)
            row0 = jj // _SPG
            col = (jj % _SPG) * 16 + iota
            w0 = wbuf[bf, 0, pl.ds(p0, 16)]
            w1 = wbuf[bf, 1, pl.ds(p0, 16)]
            w2 = wbuf[bf, 2, pl.ds(p0, 16)]
            for feat in range(2):
                e = []
                for c8 in range(8):
                    sub = sub_b[c8 * _RPC + row0, pl.ds(
                        pl.multiple_of((jj % _SPG) * 16, 16), 16)]
                    e.append(plsc.load_gather(
                        rows_b,
                        [jnp.full((16,), c8 * _RPC, jnp.int32) + row0,
                         col, sub + feat]))
                c00 = e[0] * (1 - w0) + e[4] * w0
                c01 = e[1] * (1 - w0) + e[5] * w0
                c10 = e[2] * (1 - w0) + e[6] * w0
                c11 = e[3] * (1 - w0) + e[7] * w0
                c0 = c00 * (1 - w1) + c10 * w1
                c1 = c01 * (1 - w1) + c11 * w1
                val = (c0 * (1 - w2) + c1 * w2) * gate
                out_v[2 * l + feat, pl.ds(p0, 16)] = val
            return c

        lax.fori_loop(0, _NG, body, 0, unroll=False)

    def do_chunk(ch, carry):
        base = wid * _PTS_PER_W + ch * _CHUNK
        pltpu.sync_copy(xyz_hbm.at[pl.ds(base * 3, 3 * _CHUNK)], xyz_v)
        pass_a(0, 0)
        fire(0)
        for l in range(_N_LEVELS):
            bf = l % 2
            if l + 1 < _N_LEVELS:
                pass_a(l + 1, 1 - bf)
                fire(1 - bf)
            drain(bf)
            pass_b(l, bf)
        pltpu.sync_copy(out_v, out_hbm.at[:, pl.ds(base, _CHUNK)])
        return carry

    lax.fori_loop(0, _NCHUNK, do_chunk, 0, unroll=False)


_SEL = np.zeros((4, 2, 8), np.float32)
for _c in range(4):
    for _f in range(2):
        _SEL[_c, _f, 2 * _c + _f] = 1.0


def _prep(inputs, embeddings, level_weights):
    e4 = embeddings.reshape(_N_LEVELS, _TABLE // 4, 4, 2)
    emb8 = jnp.einsum("lrcf,cfj->lrj", e4, jnp.asarray(_SEL),
                      precision=jax.lax.Precision.HIGHEST)
    return (inputs.reshape(-1),
            emb8.reshape(_N_LEVELS * _TABLE // 4, 8),
            level_weights)


@jax.jit
def _encode(xyz_flat, emb8, level_weights):
    mesh = plsc.VectorSubcoreMesh(core_axis_name="c", subcore_axis_name="s")
    f = pl.kernel(
        _sc_body,
        out_type=jax.ShapeDtypeStruct((2 * _N_LEVELS, _N_POINTS), jnp.float32),
        mesh=mesh,
        compiler_params=pltpu.CompilerParams(
            needs_layout_passes=False, use_tc_tiling_on_sc=False),
        scratch_types=_SCRATCH,
    )
    return f(xyz_flat, emb8, level_weights)


def kernel(inputs, embeddings, level_weights):
    out_t = _encode(*_prep(inputs, embeddings, level_weights))
    eye = jnp.eye(2 * _N_LEVELS, dtype=jnp.float32)
    return jax.numpy.einsum("kc,kf->cf", out_t, eye,
                            precision=jax.lax.Precision.HIGHEST)
